# Initial kernel scaffold; baseline (speedup 1.0000x reference)
import jax
import jax.numpy as jnp
from jax.experimental import pallas as pl


def kernel(x, adj_counts, inv_deg, w1l, w1r, b1, w2l, w2r, b2):
    raise NotImplementedError("write your pallas kernel here")


# speedup vs baseline: 2.0316x; 2.0316x over previous
import functools

import jax
import jax.numpy as jnp
from jax.experimental import pallas as pl
from jax.experimental.pallas import tpu as pltpu

_LANE = 128
_TILE = 512


def _round_up(v, m):
    return ((v + m - 1) // m) * m


def _pad2d(a, rows, cols):
    pr, pc = rows - a.shape[0], cols - a.shape[1]
    if pr == 0 and pc == 0:
        return a
    return jnp.pad(a, ((0, pr), (0, pc)))


def _l2norm(h, eps=1e-12):
    ss = jnp.sum(h * h, axis=-1, keepdims=True)
    return h * jax.lax.rsqrt(jnp.maximum(ss, eps * eps))


def _layer1_body(b_ref, xb_ref, xi_ref, inv_ref, w1l_ref, w1r_ref, b1_ref,
                 w2l_ref, w2r_ref, b2_ref, z2_ref, s2_ref):
    bf = b_ref[...].astype(jnp.bfloat16)
    agg = jnp.dot(bf, xb_ref[...], preferred_element_type=jnp.float32)
    agg = agg * inv_ref[...]
    h = (jnp.dot(agg, w1l_ref[...], preferred_element_type=jnp.float32)
         + jnp.dot(xi_ref[...], w1r_ref[...], preferred_element_type=jnp.float32)
         + b1_ref[...])
    h = _l2norm(h)
    h = jnp.maximum(h, 0.0)
    z2_ref[...] = jnp.dot(
        h, w2l_ref[...], preferred_element_type=jnp.float32).astype(jnp.bfloat16)
    s2_ref[...] = (jnp.dot(h, w2r_ref[...], preferred_element_type=jnp.float32)
                   + b2_ref[...])


def _layer2_body(num_classes, b_ref, z_ref, inv_ref, s_ref, out_ref):
    bf = b_ref[...].astype(jnp.bfloat16)
    o = jnp.dot(bf, z_ref[...], preferred_element_type=jnp.float32)
    o = o * inv_ref[...] + s_ref[...]
    o = _l2norm(o)
    lane = jax.lax.broadcasted_iota(jnp.int32, o.shape, 1)
    valid = lane < num_classes
    o = jnp.where(valid, o, -jnp.inf)
    m = jnp.max(o, axis=-1, keepdims=True)
    shifted = o - m
    sum_exp = jnp.sum(jnp.where(valid, jnp.exp(shifted), 0.0),
                      axis=-1, keepdims=True)
    log_probs = shifted - jnp.log(sum_exp)
    out_ref[...] = jnp.where(valid, log_probs, 0.0)


def kernel(x, adj_counts, inv_deg, w1l, w1r, b1, w2l, w2r, b2):
    n, fin = x.shape
    hid = w1l.shape[1]
    ncls = w2l.shape[1]
    f32 = jnp.float32
    bf16 = jnp.bfloat16

    half = max(_LANE, (_round_up(n, _LANE) // 2) // _LANE * _LANE)
    tile = min(_TILE, half)
    n_p = _round_up(n, tile)
    f_p = _round_up(fin, _LANE)
    h_p = _round_up(hid, _LANE)
    c_p = _round_up(ncls, _LANE)

    b_p = _pad2d(adj_counts.astype(f32), n_p, n_p)
    x_p = _pad2d(x.astype(f32), n_p, f_p)
    x_bf = x_p.astype(bf16)
    inv_p = _pad2d(inv_deg.astype(f32).reshape(-1, 1), n_p, 1)
    w1l_p = _pad2d(w1l.astype(f32), f_p, h_p)
    w1r_p = _pad2d(w1r.astype(f32), f_p, h_p)
    b1_p = _pad2d(b1.astype(f32).reshape(1, -1), 1, h_p)
    w2l_p = _pad2d(w2l.astype(f32), h_p, c_p)
    w2r_p = _pad2d(w2r.astype(f32), h_p, c_p)
    b2_p = _pad2d(b2.astype(f32).reshape(1, -1), 1, c_p)

    grid = (n_p // tile,)

    z2, s2 = pl.pallas_call(
        _layer1_body,
        out_shape=(jax.ShapeDtypeStruct((n_p, c_p), bf16),
                   jax.ShapeDtypeStruct((n_p, c_p), f32)),
        grid=grid,
        in_specs=[
            pl.BlockSpec((tile, n_p), lambda i: (i, 0)),
            pl.BlockSpec((n_p, f_p), lambda i: (0, 0)),
            pl.BlockSpec((tile, f_p), lambda i: (i, 0)),
            pl.BlockSpec((tile, 1), lambda i: (i, 0)),
            pl.BlockSpec((f_p, h_p), lambda i: (0, 0)),
            pl.BlockSpec((f_p, h_p), lambda i: (0, 0)),
            pl.BlockSpec((1, h_p), lambda i: (0, 0)),
            pl.BlockSpec((h_p, c_p), lambda i: (0, 0)),
            pl.BlockSpec((h_p, c_p), lambda i: (0, 0)),
            pl.BlockSpec((1, c_p), lambda i: (0, 0)),
        ],
        out_specs=[
            pl.BlockSpec((tile, c_p), lambda i: (i, 0)),
            pl.BlockSpec((tile, c_p), lambda i: (i, 0)),
        ],
        compiler_params=pltpu.CompilerParams(
            dimension_semantics=("parallel",),
            vmem_limit_bytes=56 * 1024 * 1024,
        ),
    )(b_p, x_bf, x_p, inv_p, w1l_p, w1r_p, b1_p, w2l_p, w2r_p, b2_p)

    out_p = pl.pallas_call(
        functools.partial(_layer2_body, ncls),
        out_shape=jax.ShapeDtypeStruct((n_p, c_p), f32),
        grid=grid,
        in_specs=[
            pl.BlockSpec((tile, n_p), lambda i: (i, 0)),
            pl.BlockSpec((n_p, c_p), lambda i: (0, 0)),
            pl.BlockSpec((tile, 1), lambda i: (i, 0)),
            pl.BlockSpec((tile, c_p), lambda i: (i, 0)),
        ],
        out_specs=pl.BlockSpec((tile, c_p), lambda i: (i, 0)),
        compiler_params=pltpu.CompilerParams(
            dimension_semantics=("parallel",),
            vmem_limit_bytes=56 * 1024 * 1024,
        ),
    )(b_p, z2, inv_p, s2)

    return out_p[:n, :ncls]
